# baseline (speedup 1.0000x reference)
import jax
import jax.numpy as jnp
import numpy as np
from jax.experimental import pallas as pl
from jax.experimental.pallas import tpu as pltpu

INPUT_SIZE = 16384
OUTPUT_SIZE = 1024
N_UNITS = 64
CHANNEL_SIZE = 64
ENV_DIMS = 32
REACH = 1.0
GARBAGE_DECAY = 0.9
REACH_THRESHOLD = float(np.clip(1.0 - REACH, -1.0, 1.0))
GARBAGE_SCALE = float(np.clip(1.0 - GARBAGE_DECAY, 0.0, 1.0))
CAPACITY = INPUT_SIZE + N_UNITS
N_QUERIES = N_UNITS + OUTPUT_SIZE

NTILES = 2
TILE = CAPACITY // NTILES
C = CHANNEL_SIZE
E = ENV_DIMS


def _normalize(x):
    s = jnp.sum(x * x, axis=1, keepdims=True)
    return x * jax.lax.rsqrt(jnp.clip(s, 1e-24, None))


def _fused_kernel(state_ref, tails_ref, eh_ref, uh_ref, w_ref, b_ref,
                  db_ref, exit_ref, gsum_ref,
                  q_ref, dist_acc, gsum_acc):
    i = pl.program_id(0)

    @pl.when(i == 0)
    def _init_queries():
        q_ref[...] = jnp.concatenate(
            [_normalize(uh_ref[...]), _normalize(eh_ref[...])], axis=0)

    base = i * TILE
    rows = jax.lax.broadcasted_iota(jnp.int32, (TILE, 1), 0) + base
    valid = rows < INPUT_SIZE

    tp = jnp.where(valid, tails_ref[...], 0.0)
    zn = _normalize(tp)
    sp = jnp.where(valid, state_ref[...], 0.0)

    q = q_ref[...]
    sims = jax.lax.dot_general(
        q, zn, (((1,), (1,)), ((), ())),
        preferred_element_type=jnp.float32)
    infl = jnp.maximum(sims - REACH_THRESHOLD, 0.0)

    dist_part = jnp.dot(infl, sp, preferred_element_type=jnp.float32)

    t = sp * (jnp.sum(infl, axis=0) - 1.0)[:, None]
    db_ref[...] = sp - GARBAGE_SCALE * t
    g_part = -jnp.sum(t, axis=0, keepdims=True)

    @pl.when(i == 0)
    def _init():
        dist_acc[...] = dist_part
        gsum_acc[...] = g_part

    @pl.when(i > 0)
    def _accum():
        dist_acc[...] += dist_part
        gsum_acc[...] += g_part

    @pl.when(i == NTILES - 1)
    def _finish():
        dist = dist_acc[...]
        exit_ref[...] = dist[N_UNITS:]
        gsum_ref[...] = gsum_acc[...]
        unit_in = dist[:N_UNITS]
        w = w_ref[...]
        prod = jnp.sum(unit_in[:, :, None] * w, axis=1)
        unit_out = unit_in + jnp.maximum(prod + b_ref[...], 0.0)
        db_ref[TILE - N_UNITS:, :] = unit_out


@jax.jit
def _run(state, input_tails, exit_heads, unit_heads, unit_W, unit_b):
    out_shapes = (
        jax.ShapeDtypeStruct((CAPACITY, C), jnp.float32),
        jax.ShapeDtypeStruct((OUTPUT_SIZE, C), jnp.float32),
        jax.ShapeDtypeStruct((1, C), jnp.float32),
    )
    db_data, exit_out, gsum = pl.pallas_call(
        _fused_kernel,
        grid=(NTILES,),
        in_specs=[
            pl.BlockSpec((TILE, C), lambda i: (i, 0)),
            pl.BlockSpec((TILE, E), lambda i: (i, 0)),
            pl.BlockSpec((OUTPUT_SIZE, E), lambda i: (0, 0)),
            pl.BlockSpec((N_UNITS, E), lambda i: (0, 0)),
            pl.BlockSpec((N_UNITS, C, C), lambda i: (0, 0, 0)),
            pl.BlockSpec((N_UNITS, C), lambda i: (0, 0)),
        ],
        out_specs=[
            pl.BlockSpec((TILE, C), lambda i: (i, 0)),
            pl.BlockSpec((OUTPUT_SIZE, C), lambda i: (0, 0)),
            pl.BlockSpec((1, C), lambda i: (0, 0)),
        ],
        out_shape=out_shapes,
        scratch_shapes=[
            pltpu.VMEM((N_QUERIES, E), jnp.float32),
            pltpu.VMEM((N_QUERIES, C), jnp.float32),
            pltpu.VMEM((1, C), jnp.float32),
        ],
    )(state, input_tails, exit_heads, unit_heads, unit_W, unit_b)
    return exit_out, gsum.reshape(C), db_data


def kernel(state, input_tails, exit_heads, unit_heads, unit_tails, unit_W,
           unit_b, step=1):
    del unit_tails, step
    return _run(state, input_tails, exit_heads, unit_heads, unit_W, unit_b)


# speedup vs baseline: 1.0142x; 1.0142x over previous
import jax
import jax.numpy as jnp
import numpy as np
from jax.experimental import pallas as pl
from jax.experimental.pallas import tpu as pltpu

INPUT_SIZE = 16384
OUTPUT_SIZE = 1024
N_UNITS = 64
CHANNEL_SIZE = 64
ENV_DIMS = 32
REACH = 1.0
GARBAGE_DECAY = 0.9
REACH_THRESHOLD = float(np.clip(1.0 - REACH, -1.0, 1.0))
GARBAGE_SCALE = float(np.clip(1.0 - GARBAGE_DECAY, 0.0, 1.0))
CAPACITY = INPUT_SIZE + N_UNITS
N_QUERIES = N_UNITS + OUTPUT_SIZE

NTILES = 4
TILE = CAPACITY // NTILES
C = CHANNEL_SIZE
E = ENV_DIMS


def _normalize(x):
    s = jnp.sum(x * x, axis=1, keepdims=True)
    return x * jax.lax.rsqrt(jnp.clip(s, 1e-24, None))


def _fused_kernel(state_ref, tails_ref, eh_ref, uh_ref, w_ref, b_ref,
                  db_ref, exit_ref, gsum_ref,
                  q_ref, dist_acc, gsum_acc):
    i = pl.program_id(0)

    @pl.when(i == 0)
    def _init_queries():
        q_ref[...] = jnp.concatenate(
            [_normalize(uh_ref[...]), _normalize(eh_ref[...])], axis=0)

    base = i * TILE
    rows = jax.lax.broadcasted_iota(jnp.int32, (TILE, 1), 0) + base
    valid = rows < INPUT_SIZE

    tp = jnp.where(valid, tails_ref[...], 0.0)
    zn = _normalize(tp)
    sp = jnp.where(valid, state_ref[...], 0.0)

    q = q_ref[...]
    sims = jax.lax.dot_general(
        q.astype(jnp.bfloat16), zn.astype(jnp.bfloat16),
        (((1,), (1,)), ((), ())),
        preferred_element_type=jnp.float32)
    if REACH_THRESHOLD == 0.0:
        infl = jnp.maximum(sims, 0.0).astype(jnp.bfloat16)
    else:
        infl = jnp.maximum(sims - REACH_THRESHOLD, 0.0).astype(jnp.bfloat16)

    dist_part = jnp.dot(infl, sp.astype(jnp.bfloat16),
                        preferred_element_type=jnp.float32)

    colsum = jnp.sum(infl, axis=0, dtype=jnp.float32)
    t = sp * (colsum - 1.0)[:, None]
    db_ref[...] = sp - GARBAGE_SCALE * t
    g_part = -jnp.sum(t, axis=0, keepdims=True)

    @pl.when(i == 0)
    def _init():
        dist_acc[...] = dist_part
        gsum_acc[...] = g_part

    @pl.when(i > 0)
    def _accum():
        dist_acc[...] += dist_part
        gsum_acc[...] += g_part

    @pl.when(i == NTILES - 1)
    def _finish():
        dist = dist_acc[...]
        exit_ref[...] = dist[N_UNITS:]
        gsum_ref[...] = gsum_acc[...]
        unit_in = dist[:N_UNITS]
        w = w_ref[...]
        prod = jnp.sum(unit_in[:, :, None] * w, axis=1)
        unit_out = unit_in + jnp.maximum(prod + b_ref[...], 0.0)
        db_ref[TILE - N_UNITS:, :] = unit_out


@jax.jit
def _run(state, input_tails, exit_heads, unit_heads, unit_W, unit_b):
    out_shapes = (
        jax.ShapeDtypeStruct((CAPACITY, C), jnp.float32),
        jax.ShapeDtypeStruct((OUTPUT_SIZE, C), jnp.float32),
        jax.ShapeDtypeStruct((1, C), jnp.float32),
    )
    db_data, exit_out, gsum = pl.pallas_call(
        _fused_kernel,
        grid=(NTILES,),
        in_specs=[
            pl.BlockSpec((TILE, C), lambda i: (i, 0)),
            pl.BlockSpec((TILE, E), lambda i: (i, 0)),
            pl.BlockSpec((OUTPUT_SIZE, E), lambda i: (0, 0)),
            pl.BlockSpec((N_UNITS, E), lambda i: (0, 0)),
            pl.BlockSpec((N_UNITS, C, C), lambda i: (0, 0, 0)),
            pl.BlockSpec((N_UNITS, C), lambda i: (0, 0)),
        ],
        out_specs=[
            pl.BlockSpec((TILE, C), lambda i: (i, 0)),
            pl.BlockSpec((OUTPUT_SIZE, C), lambda i: (0, 0)),
            pl.BlockSpec((1, C), lambda i: (0, 0)),
        ],
        out_shape=out_shapes,
        scratch_shapes=[
            pltpu.VMEM((N_QUERIES, E), jnp.float32),
            pltpu.VMEM((N_QUERIES, C), jnp.float32),
            pltpu.VMEM((1, C), jnp.float32),
        ],
    )(state, input_tails, exit_heads, unit_heads, unit_W, unit_b)
    return exit_out, gsum.reshape(C), db_data


def kernel(state, input_tails, exit_heads, unit_heads, unit_tails, unit_W,
           unit_b, step=1):
    del unit_tails, step
    return _run(state, input_tails, exit_heads, unit_heads, unit_W, unit_b)
